# baseline (speedup 1.0000x reference)
import jax
import jax.numpy as jnp
from jax.experimental import pallas as pl

_R = 64


def _gelu_exact(v):
    return 0.5 * v * (1.0 + jax.lax.erf(v * 0.7071067811865476))


def _body(xr_ref, w1a_ref, b1a_ref, w2a_ref, b2a_ref,
          w1b_ref, b1b_ref, w2b_ref, b2b_ref,
          ws1_ref, bs1_ref, ws3_ref, bs3_ref, ws2_ref, bs2_ref,
          out_ref):
    f32 = jnp.float32
    bf16 = jnp.bfloat16
    x = xr_ref[0]
    x16 = x.astype(bf16)
    h1 = jnp.dot(x16, ws1_ref[...], preferred_element_type=f32) + bs1_ref[...]
    h3 = jnp.dot(x16, ws3_ref[...], preferred_element_type=f32) + bs3_ref[...]
    g = (jax.nn.silu(h1) * h3).astype(bf16)
    shared = jnp.dot(g, ws2_ref[...], preferred_element_type=f32) + bs2_ref[...]
    ha = _gelu_exact(jnp.dot(x16, w1a_ref[0].astype(bf16),
                             preferred_element_type=f32) + b1a_ref[0])
    ea = jnp.dot(ha.astype(bf16), w2a_ref[0].astype(bf16),
                 preferred_element_type=f32) + b2a_ref[0]
    hb = _gelu_exact(jnp.dot(x16, w1b_ref[0].astype(bf16),
                             preferred_element_type=f32) + b1b_ref[0])
    eb = jnp.dot(hb.astype(bf16), w2b_ref[0].astype(bf16),
                 preferred_element_type=f32) + b2b_ref[0]
    out_ref[0] = 0.5 * shared + 0.25 * (ea + eb)


def kernel(x, t_emb, Ws1, bs1, Ws3, bs3, Ws2, bs2, W1, b1, W2, b2):
    B, T, C = x.shape
    N = B * T
    J = N // _R
    E, _, HR = W1.shape
    HS = Ws1.shape[1]
    f32 = jnp.float32

    bf16 = jnp.bfloat16
    xr = x.reshape(J, _R, C).transpose(1, 0, 2)
    Ws1c, Ws3c, Ws2c = Ws1.astype(bf16), Ws3.astype(bf16), Ws2.astype(bf16)
    b1r = b1[:, None, :]
    b2r = b2[:, None, :]
    bs1r = bs1[None, :]
    bs3r = bs3[None, :]
    bs2r = bs2[None, :]

    out = pl.pallas_call(
        _body,
        grid=(_R,),
        in_specs=[
            pl.BlockSpec((1, J, C), lambda r: (r, 0, 0)),
            pl.BlockSpec((1, C, HR), lambda r: ((3 * r) % _R, 0, 0)),
            pl.BlockSpec((1, 1, HR), lambda r: ((3 * r) % _R, 0, 0)),
            pl.BlockSpec((1, HR, C), lambda r: ((3 * r) % _R, 0, 0)),
            pl.BlockSpec((1, 1, C), lambda r: ((3 * r) % _R, 0, 0)),
            pl.BlockSpec((1, C, HR), lambda r: ((3 * r + 47) % _R, 0, 0)),
            pl.BlockSpec((1, 1, HR), lambda r: ((3 * r + 47) % _R, 0, 0)),
            pl.BlockSpec((1, HR, C), lambda r: ((3 * r + 47) % _R, 0, 0)),
            pl.BlockSpec((1, 1, C), lambda r: ((3 * r + 47) % _R, 0, 0)),
            pl.BlockSpec((C, HS), lambda r: (0, 0)),
            pl.BlockSpec((1, HS), lambda r: (0, 0)),
            pl.BlockSpec((C, HS), lambda r: (0, 0)),
            pl.BlockSpec((1, HS), lambda r: (0, 0)),
            pl.BlockSpec((HS, C), lambda r: (0, 0)),
            pl.BlockSpec((1, C), lambda r: (0, 0)),
        ],
        out_specs=pl.BlockSpec((1, J, C), lambda r: (r, 0, 0)),
        out_shape=jax.ShapeDtypeStruct((_R, J, C), f32),
    )(xr, W1, b1r, W2, b2r, W1, b1r, W2, b2r,
      Ws1c, bs1r, Ws3c, bs3r, Ws2c, bs2r)

    return out.transpose(1, 0, 2).reshape(B, T, C)


# speedup vs baseline: 1.4010x; 1.4010x over previous
import jax
import jax.numpy as jnp
from jax.experimental import pallas as pl
from jax.experimental.pallas import tpu as pltpu

_R = 64
_NSTEP = _R + 1


def _gelu_exact(v):
    return 0.5 * v * (1.0 + jax.lax.erf(v * 0.7071067811865476))


def _body(x_hbm, w1_ref, b1_ref, w2_ref, b2_ref,
          ws1_ref, bs1_ref, ws3_ref, bs3_ref, ws2_ref, bs2_ref,
          out_hbm,
          xbuf, obuf, k0buf, in_sems, out_sems):
    f32 = jnp.float32
    bf16 = jnp.bfloat16
    i = pl.program_id(0)
    c_cur = jax.lax.rem(37 * i, _R)
    c_prev = jax.lax.rem(37 * i + 27, _R)
    c_next = jax.lax.rem(37 * i + 37, _R)

    s_cur = jax.lax.rem(i, 3)
    s_prev = jax.lax.rem(i + 2, 3)
    s_next = jax.lax.rem(i + 1, 3)
    o_slot = jax.lax.rem(i, 2)

    def in_copy(c, slot):
        return pltpu.make_async_copy(
            x_hbm.at[:, c, :], xbuf.at[slot], in_sems.at[slot])

    @pl.when(i == 0)
    def _():
        in_copy(c_cur, s_cur).start()
    in_copy(c_cur, s_cur).wait()

    @pl.when(i < _NSTEP - 1)
    def _():
        in_copy(c_next, s_next).start()

    x_cur = xbuf[s_cur]
    x_prev = xbuf[s_prev]
    j = x_cur.shape[0]

    xe = jnp.concatenate([x_prev, x_cur], axis=0)
    h = _gelu_exact(jnp.dot(xe, w1_ref[0], preferred_element_type=f32)
                    + b1_ref[0])
    eo = jnp.dot(h, w2_ref[0], preferred_element_type=f32) + b2_ref[0]
    k0_prev = k0buf[...]
    k0buf[...] = eo[j:, :]

    @pl.when(i > 0)
    def _():
        x16 = x_prev.astype(bf16)
        h1 = jnp.dot(x16, ws1_ref[...], preferred_element_type=f32) + bs1_ref[...]
        h3 = jnp.dot(x16, ws3_ref[...], preferred_element_type=f32) + bs3_ref[...]
        g = (jax.nn.silu(h1) * h3).astype(bf16)
        shared = jnp.dot(g, ws2_ref[...], preferred_element_type=f32) + bs2_ref[...]

        @pl.when(i >= 3)
        def _():
            pltpu.make_async_copy(obuf.at[o_slot], out_hbm.at[:, c_prev, :],
                                  out_sems.at[o_slot]).wait()
        obuf[o_slot] = 0.5 * shared + 0.25 * (k0_prev + eo[:j, :])
        pltpu.make_async_copy(obuf.at[o_slot], out_hbm.at[:, c_prev, :],
                              out_sems.at[o_slot]).start()

    @pl.when(i == _NSTEP - 1)
    def _():
        pltpu.make_async_copy(obuf.at[o_slot], out_hbm.at[:, c_prev, :],
                              out_sems.at[o_slot]).wait()
        c_pp = jax.lax.rem(37 * i + 54, _R)
        pltpu.make_async_copy(obuf.at[1 - o_slot], out_hbm.at[:, c_pp, :],
                              out_sems.at[1 - o_slot]).wait()


def kernel(x, t_emb, Ws1, bs1, Ws3, bs3, Ws2, bs2, W1, b1, W2, b2):
    B, T, C = x.shape
    N = B * T
    J = N // _R
    E, _, HR = W1.shape
    HS = Ws1.shape[1]
    f32 = jnp.float32
    bf16 = jnp.bfloat16

    x3 = x.reshape(J, _R, C)
    b1r = b1[:, None, :]
    b2r = b2[:, None, :]
    bs1r = bs1[None, :]
    bs3r = bs3[None, :]
    bs2r = bs2[None, :]
    Ws1c, Ws3c, Ws2c = Ws1.astype(bf16), Ws3.astype(bf16), Ws2.astype(bf16)

    out = pl.pallas_call(
        _body,
        grid=(_NSTEP,),
        in_specs=[
            pl.BlockSpec(memory_space=pl.ANY),
            pl.BlockSpec((1, C, HR), lambda i: ((47 * i) % _R, 0, 0)),
            pl.BlockSpec((1, 1, HR), lambda i: ((47 * i) % _R, 0, 0)),
            pl.BlockSpec((1, HR, C), lambda i: ((47 * i) % _R, 0, 0)),
            pl.BlockSpec((1, 1, C), lambda i: ((47 * i) % _R, 0, 0)),
            pl.BlockSpec((C, HS), lambda i: (0, 0)),
            pl.BlockSpec((1, HS), lambda i: (0, 0)),
            pl.BlockSpec((C, HS), lambda i: (0, 0)),
            pl.BlockSpec((1, HS), lambda i: (0, 0)),
            pl.BlockSpec((HS, C), lambda i: (0, 0)),
            pl.BlockSpec((1, C), lambda i: (0, 0)),
        ],
        out_specs=pl.BlockSpec(memory_space=pl.ANY),
        out_shape=jax.ShapeDtypeStruct((J, _R, C), f32),
        scratch_shapes=[
            pltpu.VMEM((3, J, C), f32),
            pltpu.VMEM((2, J, C), f32),
            pltpu.VMEM((J, C), f32),
            pltpu.SemaphoreType.DMA((3,)),
            pltpu.SemaphoreType.DMA((2,)),
        ],
    )(x3, W1, b1r, W2, b2r, Ws1c, bs1r, Ws3c, bs3r, Ws2c, bs2r)

    return out.reshape(B, T, C)


# speedup vs baseline: 1.4304x; 1.0210x over previous
import jax
import jax.numpy as jnp
from jax.experimental import pallas as pl
from jax.experimental.pallas import tpu as pltpu

_R = 64
_NSTEP = _R + 1


def _gelu_exact(v):
    return 0.5 * v * (1.0 + jax.lax.erf(v * 0.7071067811865476))


def _body(x_hbm, w1_ref, b1_ref, w2_ref, b2_ref,
          ws1_ref, bs1_ref, ws3_ref, bs3_ref, ws2_ref, bs2_ref,
          out_hbm,
          xbuf, obuf, k0buf, in_sems, out_sems):
    f32 = jnp.float32
    i = pl.program_id(0)
    c_cur = jax.lax.rem(37 * i, _R)
    c_prev = jax.lax.rem(37 * i + 27, _R)
    c_next = jax.lax.rem(37 * i + 37, _R)

    s_cur = jax.lax.rem(i, 3)
    s_prev = jax.lax.rem(i + 2, 3)
    s_next = jax.lax.rem(i + 1, 3)
    o_slot = jax.lax.rem(i, 2)

    def in_copy(c, slot):
        return pltpu.make_async_copy(
            x_hbm.at[:, c, :], xbuf.at[slot], in_sems.at[slot])

    @pl.when(i == 0)
    def _():
        in_copy(c_cur, s_cur).start()
    in_copy(c_cur, s_cur).wait()

    @pl.when(i < _NSTEP - 1)
    def _():
        in_copy(c_next, s_next).start()

    x_cur = xbuf[s_cur]
    x_prev = xbuf[s_prev]
    j = x_cur.shape[0]

    xe = jnp.concatenate([x_prev, x_cur], axis=0)
    h = _gelu_exact(jnp.dot(xe, w1_ref[0], preferred_element_type=f32)
                    + b1_ref[0])
    eo = jnp.dot(h, w2_ref[0], preferred_element_type=f32) + b2_ref[0]
    k0_prev = k0buf[...]
    k0buf[...] = eo[j:, :]

    @pl.when(i > 0)
    def _():
        h1 = jnp.dot(x_prev, ws1_ref[...], preferred_element_type=f32) + bs1_ref[...]
        h3 = jnp.dot(x_prev, ws3_ref[...], preferred_element_type=f32) + bs3_ref[...]
        g = jax.nn.silu(h1) * h3
        shared = jnp.dot(g, ws2_ref[...], preferred_element_type=f32) + bs2_ref[...]

        @pl.when(i >= 3)
        def _():
            pltpu.make_async_copy(obuf.at[o_slot], out_hbm.at[:, c_prev, :],
                                  out_sems.at[o_slot]).wait()
        obuf[o_slot] = 0.5 * shared + 0.25 * (k0_prev + eo[:j, :])
        pltpu.make_async_copy(obuf.at[o_slot], out_hbm.at[:, c_prev, :],
                              out_sems.at[o_slot]).start()

    @pl.when(i == _NSTEP - 1)
    def _():
        pltpu.make_async_copy(obuf.at[o_slot], out_hbm.at[:, c_prev, :],
                              out_sems.at[o_slot]).wait()
        c_pp = jax.lax.rem(37 * i + 54, _R)
        pltpu.make_async_copy(obuf.at[1 - o_slot], out_hbm.at[:, c_pp, :],
                              out_sems.at[1 - o_slot]).wait()


def kernel(x, t_emb, Ws1, bs1, Ws3, bs3, Ws2, bs2, W1, b1, W2, b2):
    B, T, C = x.shape
    N = B * T
    J = N // _R
    E, _, HR = W1.shape
    HS = Ws1.shape[1]
    f32 = jnp.float32

    x3 = x.reshape(J, _R, C)
    b1r = b1[:, None, :]
    b2r = b2[:, None, :]
    bs1r = bs1[None, :]
    bs3r = bs3[None, :]
    bs2r = bs2[None, :]

    out = pl.pallas_call(
        _body,
        grid=(_NSTEP,),
        in_specs=[
            pl.BlockSpec(memory_space=pl.ANY),
            pl.BlockSpec((1, C, HR), lambda i: ((47 * i) % _R, 0, 0)),
            pl.BlockSpec((1, 1, HR), lambda i: ((47 * i) % _R, 0, 0)),
            pl.BlockSpec((1, HR, C), lambda i: ((47 * i) % _R, 0, 0)),
            pl.BlockSpec((1, 1, C), lambda i: ((47 * i) % _R, 0, 0)),
            pl.BlockSpec((C, HS), lambda i: (0, 0)),
            pl.BlockSpec((1, HS), lambda i: (0, 0)),
            pl.BlockSpec((C, HS), lambda i: (0, 0)),
            pl.BlockSpec((1, HS), lambda i: (0, 0)),
            pl.BlockSpec((HS, C), lambda i: (0, 0)),
            pl.BlockSpec((1, C), lambda i: (0, 0)),
        ],
        out_specs=pl.BlockSpec(memory_space=pl.ANY),
        out_shape=jax.ShapeDtypeStruct((J, _R, C), f32),
        scratch_shapes=[
            pltpu.VMEM((3, J, C), f32),
            pltpu.VMEM((2, J, C), f32),
            pltpu.VMEM((J, C), f32),
            pltpu.SemaphoreType.DMA((3,)),
            pltpu.SemaphoreType.DMA((2,)),
        ],
    )(x3, W1, b1r, W2, b2r, Ws1, bs1r, Ws3, bs3r, Ws2, bs2r)

    return out.reshape(B, T, C)


# speedup vs baseline: 1.6141x; 1.1285x over previous
import jax
import jax.numpy as jnp
from jax.experimental import pallas as pl
from jax.experimental.pallas import tpu as pltpu

_R = 64
_NG = _R // 2 + 1


def _gelu_exact(v):
    return 0.5 * v * (1.0 + jax.lax.erf(v * 0.7071067811865476))


def _body(x_hbm, w1a_ref, b1a_ref, w2a_ref, b2a_ref,
          w1b_ref, b1b_ref, w2b_ref, b2b_ref,
          ws1_ref, bs1_ref, ws3_ref, bs3_ref, ws2_ref, bs2_ref,
          out_hbm,
          xbuf, obufA, obufB, k0buf, in_sems, oa_sems, ob_sems):
    f32 = jnp.float32
    g = pl.program_id(0)
    ca_prev = jax.lax.rem(10 * g + 27, _R)
    ca = jax.lax.rem(10 * g, _R)
    cb = jax.lax.rem(10 * g + 37, _R)
    sa_prev = jax.lax.rem(2 * g + 4, 5)
    sa = jax.lax.rem(2 * g, 5)
    sb = jax.lax.rem(2 * g + 1, 5)
    q = jax.lax.rem(g, 2)

    def in_copy(c, slot):
        return pltpu.make_async_copy(
            x_hbm.at[:, c, :], xbuf.at[slot], in_sems.at[slot])

    @pl.when(g == 0)
    def _():
        in_copy(ca, sa).start()
        in_copy(cb, sb).start()
    in_copy(ca, sa).wait()
    in_copy(cb, sb).wait()

    @pl.when(g < _NG - 1)
    def _():
        in_copy(jax.lax.rem(10 * g + 10, _R), jax.lax.rem(2 * g + 2, 5)).start()
        in_copy(jax.lax.rem(10 * g + 47, _R), jax.lax.rem(2 * g + 3, 5)).start()

    xa = xbuf[sa_prev]
    xb = xbuf[sa]
    xc = xbuf[sb]
    j = xb.shape[0]

    xeA = jnp.concatenate([xa, xb], axis=0)
    hA = _gelu_exact(jnp.dot(xeA, w1a_ref[0], preferred_element_type=f32)
                     + b1a_ref[0])
    eA = jnp.dot(hA, w2a_ref[0], preferred_element_type=f32) + b2a_ref[0]
    xeB = jnp.concatenate([xb, xc], axis=0)
    hB = _gelu_exact(jnp.dot(xeB, w1b_ref[0], preferred_element_type=f32)
                     + b1b_ref[0])
    eB = jnp.dot(hB, w2b_ref[0], preferred_element_type=f32) + b2b_ref[0]

    h1 = jnp.dot(xeA, ws1_ref[...], preferred_element_type=f32) + bs1_ref[...]
    h3 = jnp.dot(xeA, ws3_ref[...], preferred_element_type=f32) + bs3_ref[...]
    sh = jnp.dot(jax.nn.silu(h1) * h3, ws2_ref[...],
                 preferred_element_type=f32) + bs2_ref[...]

    k0_prev = k0buf[...]
    k0buf[...] = eB[j:, :]

    @pl.when(g > 0)
    def _():
        @pl.when(g >= 3)
        def _():
            pltpu.make_async_copy(obufA.at[q], out_hbm.at[:, ca_prev, :],
                                  oa_sems.at[q]).wait()
        obufA[q] = 0.5 * sh[:j, :] + 0.25 * (k0_prev + eA[:j, :])
        pltpu.make_async_copy(obufA.at[q], out_hbm.at[:, ca_prev, :],
                              oa_sems.at[q]).start()

    @pl.when(g >= 2)
    def _():
        pltpu.make_async_copy(obufB.at[q], out_hbm.at[:, ca, :],
                              ob_sems.at[q]).wait()
    obufB[q] = 0.5 * sh[j:, :] + 0.25 * (eA[j:, :] + eB[:j, :])
    pltpu.make_async_copy(obufB.at[q], out_hbm.at[:, ca, :],
                          ob_sems.at[q]).start()

    @pl.when(g == _NG - 1)
    def _():
        ca_prev1 = jax.lax.rem(10 * g + 17, _R)
        ca1 = jax.lax.rem(10 * g + 54, _R)
        pltpu.make_async_copy(obufA.at[q], out_hbm.at[:, ca_prev, :],
                              oa_sems.at[q]).wait()
        pltpu.make_async_copy(obufA.at[1 - q], out_hbm.at[:, ca_prev1, :],
                              oa_sems.at[1 - q]).wait()
        pltpu.make_async_copy(obufB.at[q], out_hbm.at[:, ca, :],
                              ob_sems.at[q]).wait()
        pltpu.make_async_copy(obufB.at[1 - q], out_hbm.at[:, ca1, :],
                              ob_sems.at[1 - q]).wait()


def kernel(x, t_emb, Ws1, bs1, Ws3, bs3, Ws2, bs2, W1, b1, W2, b2):
    B, T, C = x.shape
    N = B * T
    J = N // _R
    E, _, HR = W1.shape
    HS = Ws1.shape[1]
    f32 = jnp.float32

    x3 = x.reshape(J, _R, C)
    b1r = b1[:, None, :]
    b2r = b2[:, None, :]
    bs1r = bs1[None, :]
    bs3r = bs3[None, :]
    bs2r = bs2[None, :]

    out = pl.pallas_call(
        _body,
        grid=(_NG,),
        in_specs=[
            pl.BlockSpec(memory_space=pl.ANY),
            pl.BlockSpec((1, C, HR), lambda g: ((30 * g) % _R, 0, 0)),
            pl.BlockSpec((1, 1, HR), lambda g: ((30 * g) % _R, 0, 0)),
            pl.BlockSpec((1, HR, C), lambda g: ((30 * g) % _R, 0, 0)),
            pl.BlockSpec((1, 1, C), lambda g: ((30 * g) % _R, 0, 0)),
            pl.BlockSpec((1, C, HR), lambda g: ((30 * g + 47) % _R, 0, 0)),
            pl.BlockSpec((1, 1, HR), lambda g: ((30 * g + 47) % _R, 0, 0)),
            pl.BlockSpec((1, HR, C), lambda g: ((30 * g + 47) % _R, 0, 0)),
            pl.BlockSpec((1, 1, C), lambda g: ((30 * g + 47) % _R, 0, 0)),
            pl.BlockSpec((C, HS), lambda g: (0, 0)),
            pl.BlockSpec((1, HS), lambda g: (0, 0)),
            pl.BlockSpec((C, HS), lambda g: (0, 0)),
            pl.BlockSpec((1, HS), lambda g: (0, 0)),
            pl.BlockSpec((HS, C), lambda g: (0, 0)),
            pl.BlockSpec((1, C), lambda g: (0, 0)),
        ],
        out_specs=pl.BlockSpec(memory_space=pl.ANY),
        out_shape=jax.ShapeDtypeStruct((J, _R, C), f32),
        scratch_shapes=[
            pltpu.VMEM((5, J, C), f32),
            pltpu.VMEM((2, J, C), f32),
            pltpu.VMEM((2, J, C), f32),
            pltpu.VMEM((J, C), f32),
            pltpu.SemaphoreType.DMA((5,)),
            pltpu.SemaphoreType.DMA((2,)),
            pltpu.SemaphoreType.DMA((2,)),
        ],
    )(x3, W1, b1r, W2, b2r, W1, b1r, W2, b2r,
      Ws1, bs1r, Ws3, bs3r, Ws2, bs2r)

    return out.reshape(B, T, C)
